# baseline (speedup 1.0000x reference)
import functools

import jax
import jax.numpy as jnp
from jax import lax
from jax.experimental import pallas as pl
from jax.experimental.pallas import tpu as pltpu
from jax.experimental.pallas import tpu_sc as plsc

DIM = 128
NGRAPH = 64
NWORKERS = 32
CHUNK = 80
GD = NGRAPH * DIM


def _mm_body(x_ref, wt_ref, b_ref, o_ref):
    o_ref[...] = (
        jnp.dot(x_ref[...], wt_ref[...], preferred_element_type=jnp.float32)
        + b_ref[...]
    )


def _linear(x, wt, b2):
    n = x.shape[0]
    blk = 1000
    return pl.pallas_call(
        _mm_body,
        grid=(n // blk,),
        in_specs=[
            pl.BlockSpec((blk, DIM), lambda i: (i, 0)),
            pl.BlockSpec((DIM, DIM), lambda i: (0, 0)),
            pl.BlockSpec((1, DIM), lambda i: (0, 0)),
        ],
        out_specs=pl.BlockSpec((blk, DIM), lambda i: (i, 0)),
        out_shape=jax.ShapeDtypeStruct((n, DIM), jnp.float32),
    )(x, wt, b2)


def _lane_bcast(v, lane):
    return lax.gather(
        v,
        jnp.full((16, 1), lane, jnp.int32),
        lax.GatherDimensionNumbers(
            offset_dims=(), collapsed_slice_dims=(0,), start_index_map=(0,)
        ),
        (1,),
        mode=lax.GatherScatterMode.PROMISE_IN_BOUNDS,
    )


def _make_sc_edge_kernel(n_edges):
    epw = n_edges // NWORKERS
    n_chunks = epw // CHUNK
    quads = CHUNK // 4
    groups = CHUNK // 16
    mesh = plsc.VectorSubcoreMesh(core_axis_name="c", subcore_axis_name="s")

    @functools.partial(
        pl.kernel,
        out_type=(
            jax.ShapeDtypeStruct((NWORKERS, GD), jnp.float32),
            jax.ShapeDtypeStruct((NWORKERS, GD), jnp.float32),
        ),
        mesh=mesh,
        compiler_params=pltpu.CompilerParams(needs_layout_passes=False),
        scratch_types=[
            pltpu.VMEM((epw,), jnp.int32),
            pltpu.VMEM((epw,), jnp.int32),
            pltpu.VMEM((CHUNK, DIM), jnp.float32),
            pltpu.VMEM((CHUNK, DIM), jnp.float32),
            pltpu.VMEM((CHUNK,), jnp.int32),
            pltpu.VMEM((CHUNK, DIM), jnp.float32),
            pltpu.VMEM((CHUNK, DIM), jnp.float32),
            pltpu.VMEM((CHUNK,), jnp.int32),
            pltpu.VMEM((CHUNK, DIM), jnp.float32),
            pltpu.VMEM((CHUNK, 16), jnp.float32),
            pltpu.VMEM((CHUNK,), jnp.float32),
            pltpu.VMEM((GD,), jnp.float32),
            pltpu.VMEM((GD,), jnp.float32),
            pltpu.VMEM((GD,), jnp.float32),
            pltpu.VMEM((GD,), jnp.float32),
            pltpu.VMEM((GD,), jnp.float32),
            pltpu.VMEM((DIM,), jnp.float32),
            pltpu.VMEM((DIM,), jnp.float32),
            pltpu.VMEM((16,), jnp.float32),
            pltpu.SemaphoreType.DMA,
            pltpu.SemaphoreType.DMA,
        ],
    )
    def sc_edge(h_hbm, src_hbm, dst_hbm, batch_hbm, attn_hbm, ww_hbm, wb_hbm,
                out_sum, out_max,
                idx_all_s, idx_all_d,
                rows_s0, rows_d0, g0, rows_s1, rows_d1, g1,
                ar_buf, sc_buf, sig_buf, acc_sum, acc_max0, acc_max1,
                acc_max2, acc_max3,
                attn_v, ww_v, wb_v, sem0, sem1):
        wid = lax.axis_index("s") * 2 + lax.axis_index("c")
        base = wid * epw

        pltpu.sync_copy(attn_hbm, attn_v)
        pltpu.sync_copy(ww_hbm, ww_v)
        pltpu.sync_copy(wb_hbm, wb_v)
        pltpu.sync_copy(src_hbm.at[pl.ds(base, epw)], idx_all_s)
        pltpu.sync_copy(dst_hbm.at[pl.ds(base, epw)], idx_all_d)

        zero16 = jnp.zeros((16,), jnp.float32)
        ninf16 = jnp.full((16,), -jnp.inf, jnp.float32)

        def init_body(i, _):
            sl = pl.ds(i * 16, 16)
            acc_sum[sl] = zero16
            acc_max0[sl] = ninf16
            acc_max1[sl] = ninf16
            acc_max2[sl] = ninf16
            acc_max3[sl] = ninf16
            return 0

        lax.fori_loop(0, GD // 16, init_body, 0)

        attn_r = [attn_v[pl.ds(16 * j, 16)] for j in range(8)]
        ww_r = [ww_v[pl.ds(16 * j, 16)] for j in range(8)]
        wb_vec = wb_v[...]
        iota16 = lax.iota(jnp.int32, 16)

        def fire(ci, rows_s, rows_d, g, sem):
            isl = idx_all_s.at[pl.ds(ci * CHUNK, CHUNK)]
            idl = idx_all_d.at[pl.ds(ci * CHUNK, CHUNK)]
            pltpu.async_copy(h_hbm.at[isl], rows_s, sem)
            pltpu.async_copy(h_hbm.at[idl], rows_d, sem)
            pltpu.async_copy(batch_hbm.at[isl], g, sem)

        def drain(rows_s, rows_d, g, sem):
            pltpu.make_async_copy(h_hbm.at[pl.ds(0, CHUNK)], rows_s, sem).wait()
            pltpu.make_async_copy(h_hbm.at[pl.ds(0, CHUNK)], rows_d, sem).wait()
            pltpu.make_async_copy(batch_hbm.at[pl.ds(0, CHUNK)], g, sem).wait()

        def compute(rows_s, rows_d, g):
            def a_edge(i):
                sc = zero16
                for j in range(8):
                    js = pl.ds(16 * j, 16)
                    s = rows_s[i, js]
                    d = rows_d[i, js]
                    v = s + d
                    xe = jnp.where(v > 0, v, jnp.exp(v) - 1.0)
                    m = xe * attn_r[j]
                    ar = jnp.where(m > 0, m, jnp.exp(m) - 1.0)
                    ar_buf[i, js] = ar
                    sc = sc + ar * ww_r[j]
                sc_buf[i, :] = sc

            plsc.parallel_loop(0, CHUNK, unroll=4)(a_edge)

            def score_body(gr, _):
                i0 = gr * 16
                idx0 = i0 + iota16
                cols = [
                    plsc.load_gather(
                        sc_buf, [idx0, jnp.full((16,), l, jnp.int32)]
                    )
                    for l in range(16)
                ]
                t01 = (cols[0] + cols[1]) + (cols[2] + cols[3])
                t23 = (cols[4] + cols[5]) + (cols[6] + cols[7])
                t45 = (cols[8] + cols[9]) + (cols[10] + cols[11])
                t67 = (cols[12] + cols[13]) + (cols[14] + cols[15])
                tot = (t01 + t23) + (t45 + t67)
                z = tot + wb_vec
                sig_buf[pl.ds(i0, 16)] = 1.0 / (1.0 + jnp.exp(-z))
                return 0

            lax.fori_loop(0, groups, score_body, 0)

            def b_body(gr, _):
                i0 = gr * 16
                g16 = g[pl.ds(i0, 16)] * DIM
                sig16 = sig_buf[pl.ds(i0, 16)]
                gbs = [g16[l] for l in range(16)]
                banks = [acc_max0, acc_max1, acc_max2, acc_max3]
                for l in range(16):
                    i = i0 + l
                    gb = gbs[l]
                    sigv = _lane_bcast(sig16, l)
                    accm = banks[l % 4]
                    ars = [ar_buf[i, pl.ds(16 * j, 16)] for j in range(8)]
                    mxs = [accm[pl.ds(gb + 16 * j, 16)] for j in range(8)]
                    for j in range(8):
                        sl = pl.ds(gb + 16 * j, 16)
                        plsc.addupdate(acc_sum.at[sl], ars[j] * sigv)
                        accm[sl] = jnp.maximum(mxs[j], ars[j])
                return 0

            lax.fori_loop(0, groups, b_body, 0)

        fire(0, rows_s0, rows_d0, g0, sem0)

        def pair_body(h2, _):
            c0 = h2 * 2
            c1 = c0 + 1
            c2 = c0 + 2

            @pl.when(c1 < n_chunks)
            def _():
                fire(c1, rows_s1, rows_d1, g1, sem1)

            drain(rows_s0, rows_d0, g0, sem0)
            compute(rows_s0, rows_d0, g0)

            @pl.when(c2 < n_chunks)
            def _():
                fire(c2, rows_s0, rows_d0, g0, sem0)

            @pl.when(c1 < n_chunks)
            def _():
                drain(rows_s1, rows_d1, g1, sem1)
                compute(rows_s1, rows_d1, g1)

            return 0

        lax.fori_loop(0, (n_chunks + 1) // 2, pair_body, 0)

        def fin_body(i, _):
            sl = pl.ds(i * 16, 16)
            m01 = jnp.maximum(acc_max0[sl], acc_max1[sl])
            m23 = jnp.maximum(acc_max2[sl], acc_max3[sl])
            acc_max0[sl] = jnp.maximum(m01, m23)
            return 0

        lax.fori_loop(0, GD // 16, fin_body, 0)
        pltpu.sync_copy(acc_sum, out_sum.at[wid])
        pltpu.sync_copy(acc_max0, out_max.at[wid])

    return sc_edge


def _combine_body(ps_ref, pm_ref, o_ref):
    o_ref[:, :DIM] = jnp.sum(ps_ref[...], axis=0)
    o_ref[:, DIM:] = jnp.max(pm_ref[...], axis=0)


def _combine(psum, pmax):
    return pl.pallas_call(
        _combine_body,
        out_shape=jax.ShapeDtypeStruct((NGRAPH, 2 * DIM), jnp.float32),
    )(psum, pmax)


def kernel(x, edges_0, batch, W, b, attn, Ww, wb):
    n_edges = edges_0.shape[0]
    h = _linear(x, W.T, b.reshape(1, DIM))
    src = edges_0[:, 0]
    dst = edges_0[:, 1]
    attn_f = attn.reshape(DIM)
    ww_f = Ww.reshape(DIM)
    wb16 = jnp.broadcast_to(wb, (16,))
    psum, pmax = _make_sc_edge_kernel(n_edges)(
        h, src, dst, batch, attn_f, ww_f, wb16
    )
    return _combine(
        psum.reshape(NWORKERS, NGRAPH, DIM), pmax.reshape(NWORKERS, NGRAPH, DIM)
    )


# speedup vs baseline: 1.1632x; 1.1632x over previous
import functools

import jax
import jax.numpy as jnp
from jax import lax
from jax.experimental import pallas as pl
from jax.experimental.pallas import tpu as pltpu
from jax.experimental.pallas import tpu_sc as plsc

DIM = 128
NGRAPH = 64
NWORKERS = 32
CHUNK = 80
GD = NGRAPH * DIM


def _mm_body(x_ref, wt_ref, b_ref, o_ref):
    o_ref[...] = (
        jnp.dot(x_ref[...], wt_ref[...], preferred_element_type=jnp.float32)
        + b_ref[...]
    )


def _linear(x, wt, b2):
    n = x.shape[0]
    blk = 1000
    return pl.pallas_call(
        _mm_body,
        grid=(n // blk,),
        in_specs=[
            pl.BlockSpec((blk, DIM), lambda i: (i, 0)),
            pl.BlockSpec((DIM, DIM), lambda i: (0, 0)),
            pl.BlockSpec((1, DIM), lambda i: (0, 0)),
        ],
        out_specs=pl.BlockSpec((blk, DIM), lambda i: (i, 0)),
        out_shape=jax.ShapeDtypeStruct((n, DIM), jnp.float32),
    )(x, wt, b2)


def _lane_bcast(v, lane):
    return lax.gather(
        v,
        jnp.full((16, 1), lane, jnp.int32),
        lax.GatherDimensionNumbers(
            offset_dims=(), collapsed_slice_dims=(0,), start_index_map=(0,)
        ),
        (1,),
        mode=lax.GatherScatterMode.PROMISE_IN_BOUNDS,
    )


def _make_sc_edge_kernel(n_edges):
    epw = n_edges // NWORKERS
    n_chunks = epw // CHUNK
    quads = CHUNK // 4
    groups = CHUNK // 16
    mesh = plsc.VectorSubcoreMesh(core_axis_name="c", subcore_axis_name="s")

    @functools.partial(
        pl.kernel,
        out_type=(
            jax.ShapeDtypeStruct((NWORKERS, GD), jnp.float32),
            jax.ShapeDtypeStruct((NWORKERS, GD), jnp.float32),
        ),
        mesh=mesh,
        compiler_params=pltpu.CompilerParams(needs_layout_passes=False),
        scratch_types=[
            pltpu.VMEM((epw,), jnp.int32),
            pltpu.VMEM((epw,), jnp.int32),
            pltpu.VMEM((CHUNK, DIM), jnp.float32),
            pltpu.VMEM((CHUNK, DIM), jnp.float32),
            pltpu.VMEM((CHUNK,), jnp.int32),
            pltpu.VMEM((CHUNK, DIM), jnp.float32),
            pltpu.VMEM((CHUNK, DIM), jnp.float32),
            pltpu.VMEM((CHUNK,), jnp.int32),
            pltpu.VMEM((CHUNK, DIM), jnp.float32),
            pltpu.VMEM((CHUNK, 16), jnp.float32),
            pltpu.VMEM((CHUNK,), jnp.float32),
            pltpu.VMEM((GD,), jnp.float32),
            pltpu.VMEM((GD,), jnp.float32),
            pltpu.VMEM((GD,), jnp.float32),
            pltpu.VMEM((GD,), jnp.float32),
            pltpu.VMEM((GD,), jnp.float32),
            pltpu.VMEM((DIM,), jnp.float32),
            pltpu.VMEM((DIM,), jnp.float32),
            pltpu.VMEM((16,), jnp.float32),
            pltpu.SemaphoreType.DMA,
            pltpu.SemaphoreType.DMA,
        ],
    )
    def sc_edge(h_hbm, src_hbm, dst_hbm, batch_hbm, attn_hbm, ww_hbm, wb_hbm,
                out_sum, out_max,
                idx_all_s, idx_all_d,
                rows_s0, rows_d0, g0, rows_s1, rows_d1, g1,
                ar_buf, sc_buf, sig_buf, acc_sum, acc_max0, acc_max1,
                acc_max2, acc_max3,
                attn_v, ww_v, wb_v, sem0, sem1):
        wid = lax.axis_index("s") * 2 + lax.axis_index("c")
        base = wid * epw

        pltpu.sync_copy(attn_hbm, attn_v)
        pltpu.sync_copy(ww_hbm, ww_v)
        pltpu.sync_copy(wb_hbm, wb_v)
        pltpu.sync_copy(src_hbm.at[pl.ds(base, epw)], idx_all_s)
        pltpu.sync_copy(dst_hbm.at[pl.ds(base, epw)], idx_all_d)

        zero16 = jnp.zeros((16,), jnp.float32)
        ninf16 = jnp.full((16,), -jnp.inf, jnp.float32)

        def init_body(i, _):
            sl = pl.ds(i * 16, 16)
            acc_sum[sl] = zero16
            acc_max0[sl] = ninf16
            acc_max1[sl] = ninf16
            acc_max2[sl] = ninf16
            acc_max3[sl] = ninf16
            return 0

        lax.fori_loop(0, GD // 16, init_body, 0)

        attn_r = [attn_v[pl.ds(16 * j, 16)] for j in range(8)]
        ww_r = [ww_v[pl.ds(16 * j, 16)] for j in range(8)]
        wb_vec = wb_v[...]
        iota16 = lax.iota(jnp.int32, 16)

        def fire(ci, rows_s, rows_d, g, sem):
            isl = idx_all_s.at[pl.ds(ci * CHUNK, CHUNK)]
            idl = idx_all_d.at[pl.ds(ci * CHUNK, CHUNK)]
            pltpu.async_copy(h_hbm.at[isl], rows_s, sem)
            pltpu.async_copy(h_hbm.at[idl], rows_d, sem)
            pltpu.async_copy(batch_hbm.at[isl], g, sem)

        def drain(rows_s, rows_d, g, sem):
            pltpu.make_async_copy(h_hbm.at[pl.ds(0, CHUNK)], rows_s, sem).wait()
            pltpu.make_async_copy(h_hbm.at[pl.ds(0, CHUNK)], rows_d, sem).wait()
            pltpu.make_async_copy(batch_hbm.at[pl.ds(0, CHUNK)], g, sem).wait()

        def compute(rows_s, rows_d, g):
            def pair_a_body(q, _):
                for l in range(2):
                    i = q * 2 + l
                    sc = zero16
                    for j in range(8):
                        js = pl.ds(16 * j, 16)
                        s = rows_s[i, js]
                        d = rows_d[i, js]
                        v = s + d
                        xe = jnp.where(v > 0, v, jnp.exp(v) - 1.0)
                        m = xe * attn_r[j]
                        ar = jnp.where(m > 0, m, jnp.exp(m) - 1.0)
                        ar_buf[i, js] = ar
                        sc = sc + ar * ww_r[j]
                    sc_buf[i, :] = sc
                return 0

            lax.fori_loop(0, quads * 2, pair_a_body, 0)

            def score_body(gr, _):
                i0 = gr * 16
                idx0 = i0 + iota16
                cols = [
                    plsc.load_gather(
                        sc_buf, [idx0, jnp.full((16,), l, jnp.int32)]
                    )
                    for l in range(16)
                ]
                t01 = (cols[0] + cols[1]) + (cols[2] + cols[3])
                t23 = (cols[4] + cols[5]) + (cols[6] + cols[7])
                t45 = (cols[8] + cols[9]) + (cols[10] + cols[11])
                t67 = (cols[12] + cols[13]) + (cols[14] + cols[15])
                tot = (t01 + t23) + (t45 + t67)
                z = tot + wb_vec
                sig_buf[pl.ds(i0, 16)] = 1.0 / (1.0 + jnp.exp(-z))
                return 0

            lax.fori_loop(0, groups, score_body, 0)

            def b_body(gr, _):
                i0 = gr * 16
                g16 = g[pl.ds(i0, 16)] * DIM
                sig16 = sig_buf[pl.ds(i0, 16)]
                gbs = [g16[l] for l in range(16)]
                banks = [acc_max0, acc_max1, acc_max2, acc_max3]
                for l in range(16):
                    i = i0 + l
                    gb = gbs[l]
                    sigv = _lane_bcast(sig16, l)
                    accm = banks[l % 4]
                    ars = [ar_buf[i, pl.ds(16 * j, 16)] for j in range(8)]
                    mxs = [accm[pl.ds(gb + 16 * j, 16)] for j in range(8)]
                    for j in range(8):
                        sl = pl.ds(gb + 16 * j, 16)
                        plsc.addupdate(acc_sum.at[sl], ars[j] * sigv)
                        accm[sl] = jnp.maximum(mxs[j], ars[j])
                return 0

            lax.fori_loop(0, groups, b_body, 0)

        fire(0, rows_s0, rows_d0, g0, sem0)

        def pair_body(h2, _):
            c0 = h2 * 2
            c1 = c0 + 1
            c2 = c0 + 2

            @pl.when(c1 < n_chunks)
            def _():
                fire(c1, rows_s1, rows_d1, g1, sem1)

            drain(rows_s0, rows_d0, g0, sem0)
            compute(rows_s0, rows_d0, g0)

            @pl.when(c2 < n_chunks)
            def _():
                fire(c2, rows_s0, rows_d0, g0, sem0)

            @pl.when(c1 < n_chunks)
            def _():
                drain(rows_s1, rows_d1, g1, sem1)
                compute(rows_s1, rows_d1, g1)

            return 0

        lax.fori_loop(0, (n_chunks + 1) // 2, pair_body, 0)

        def fin_body(i, _):
            sl = pl.ds(i * 16, 16)
            m01 = jnp.maximum(acc_max0[sl], acc_max1[sl])
            m23 = jnp.maximum(acc_max2[sl], acc_max3[sl])
            acc_max0[sl] = jnp.maximum(m01, m23)
            return 0

        lax.fori_loop(0, GD // 16, fin_body, 0)
        pltpu.sync_copy(acc_sum, out_sum.at[wid])
        pltpu.sync_copy(acc_max0, out_max.at[wid])

    return sc_edge


def _combine_body(ps_ref, pm_ref, o_ref):
    o_ref[:, :DIM] = jnp.sum(ps_ref[...], axis=0)
    o_ref[:, DIM:] = jnp.max(pm_ref[...], axis=0)


def _combine(psum, pmax):
    return pl.pallas_call(
        _combine_body,
        out_shape=jax.ShapeDtypeStruct((NGRAPH, 2 * DIM), jnp.float32),
    )(psum, pmax)


def kernel(x, edges_0, batch, W, b, attn, Ww, wb):
    n_edges = edges_0.shape[0]
    h = _linear(x, W.T, b.reshape(1, DIM))
    src = edges_0[:, 0]
    dst = edges_0[:, 1]
    attn_f = attn.reshape(DIM)
    ww_f = Ww.reshape(DIM)
    wb16 = jnp.broadcast_to(wb, (16,))
    psum, pmax = _make_sc_edge_kernel(n_edges)(
        h, src, dst, batch, attn_f, ww_f, wb16
    )
    return _combine(
        psum.reshape(NWORKERS, NGRAPH, DIM), pmax.reshape(NWORKERS, NGRAPH, DIM)
    )
